# baseline (speedup 1.0000x reference)
import jax
import jax.numpy as jnp
from jax.experimental import pallas as pl

_BQ = 1024


def _min_dist_kernel(x_ref, ba_ref, o_ref):
    xx = x_ref[...]
    x0 = xx[:, 0:1]
    x1 = xx[:, 1:2]
    x2 = x0 * x0 + x1 * x1
    xa = jnp.concatenate([-2.0 * x0, -2.0 * x1, x2, jnp.ones_like(x2)], axis=1)
    t = jax.lax.dot_general(
        xa, ba_ref[...], (((1,), (0,)), ((), ())),
        preferred_element_type=jnp.float32,
        precision=jax.lax.Precision.HIGHEST)
    md2 = jnp.min(t, axis=1, keepdims=True)
    o_ref[...] = jnp.sqrt(jnp.maximum(md2, 0.0))


@jax.jit
def kernel(x, boundary_points):
    q = x.shape[0]
    b0 = boundary_points[:, 0]
    b1 = boundary_points[:, 1]
    ba = jnp.stack([b0, b1, jnp.ones_like(b0), b0 * b0 + b1 * b1])
    min_dist = pl.pallas_call(
        _min_dist_kernel,
        grid=(q // _BQ,),
        in_specs=[
            pl.BlockSpec((_BQ, 2), lambda i: (i, 0)),
            pl.BlockSpec(ba.shape, lambda i: (0, 0)),
        ],
        out_specs=pl.BlockSpec((_BQ, 1), lambda i: (i, 0)),
        out_shape=jax.ShapeDtypeStruct((q, 1), x.dtype),
    )(x, ba)
    return jnp.concatenate([x, min_dist], axis=-1)


# speedup vs baseline: 3.4123x; 3.4123x over previous
import jax
import jax.numpy as jnp
from jax.experimental import pallas as pl

_BQ = 2048


def _min_dist_kernel(x_ref, brow_ref, o_ref):
    xx = x_ref[...]
    qx = xx[:, 0:1]
    qy = xx[:, 1:2]
    qxm2 = -2.0 * qx
    qym2 = -2.0 * qy
    bx = brow_ref[0:1, :]
    by = brow_ref[1:2, :]
    b2 = brow_ref[2:3, :]
    t = (qxm2 * bx + qym2 * by) + b2
    md = jnp.min(t, axis=1, keepdims=True) + (qx * qx + qy * qy)
    o_ref[...] = jnp.sqrt(jnp.maximum(md, 0.0))


@jax.jit
def kernel(x, boundary_points):
    q = x.shape[0]
    bx = boundary_points[:, 0]
    by = boundary_points[:, 1]
    brow = jnp.stack([bx, by, bx * bx + by * by])
    min_dist = pl.pallas_call(
        _min_dist_kernel,
        grid=(q // _BQ,),
        in_specs=[
            pl.BlockSpec((_BQ, 2), lambda i: (i, 0)),
            pl.BlockSpec(brow.shape, lambda i: (0, 0)),
        ],
        out_specs=pl.BlockSpec((_BQ, 1), lambda i: (i, 0)),
        out_shape=jax.ShapeDtypeStruct((q, 1), x.dtype),
    )(x, brow)
    return jnp.concatenate([x, min_dist], axis=-1)
